# baseline (speedup 1.0000x reference)
import functools

import jax
import jax.numpy as jnp
from jax.experimental import pallas as pl
from jax.experimental.pallas import tpu as pltpu

_NUM_CLASSES = 400
_LABEL_SMOOTH = 0.1
_ALPHA = 1.0
_IGNORE_CLS = -1
_B = 16


def _mix_x_body(partner_ref, lam_ref, x_ref, o_ref):
    for i in range(_B):
        lam = lam_ref[i]
        p = partner_ref[i]
        xi = x_ref[pl.ds(i, 1)]
        xp = x_ref[pl.ds(p, 1)]
        o_ref[pl.ds(i, 1)] = xi * lam + xp * (1.0 - lam)


def _mix_x(x, partner, lam_rows, tchunk):
    b, c, t, h, w = x.shape
    grid_spec = pltpu.PrefetchScalarGridSpec(
        num_scalar_prefetch=2,
        grid=(c, t // tchunk),
        in_specs=[
            pl.BlockSpec((b, 1, tchunk, h, w),
                         lambda j, k, p, l: (0, j, k, 0, 0)),
        ],
        out_specs=pl.BlockSpec((b, 1, tchunk, h, w),
                               lambda j, k, p, l: (0, j, k, 0, 0)),
    )
    return pl.pallas_call(
        _mix_x_body,
        grid_spec=grid_spec,
        out_shape=jax.ShapeDtypeStruct(x.shape, jnp.float32),
        compiler_params=pltpu.CompilerParams(
            dimension_semantics=("arbitrary", "arbitrary"),
        ),
    )(partner, lam_rows, x)


def _one_hot_smooth(t):
    nt = _LABEL_SMOOTH / _NUM_CLASSES
    tv = 1.0 - _LABEL_SMOOTH + nt
    hot = jax.nn.one_hot(jnp.squeeze(t, axis=-1), _NUM_CLASSES, dtype=jnp.float32)
    return jnp.where(hot > 0.5, jnp.float32(tv), jnp.float32(nt))


def kernel(x_video_rgb, labels_action, labels_subclips_action):
    ts = jnp.squeeze(labels_subclips_action, axis=-1)
    mask = jnp.all(ts != _IGNORE_CLS, axis=-1)
    k = jnp.sum(mask)
    no_mix = k <= 1
    order = jnp.argsort(jnp.logical_not(mask), stable=True)
    rank = jnp.cumsum(mask) - 1
    partner = order[jnp.clip(k - 1 - rank, 0, _B - 1)].astype(jnp.int32)
    lam = jax.random.beta(jax.random.key(1), _ALPHA, _ALPHA)
    mix_on = mask & jnp.logical_not(no_mix)
    lam_rows = jnp.where(mix_on, lam, 1.0).astype(jnp.float32)

    x_out = _mix_x(x_video_rgb, partner, lam_rows, tchunk=1)

    labels_out = _one_hot_smooth(labels_action)
    subclips_ignore_index = labels_subclips_action == _IGNORE_CLS
    val_tmp = jnp.where(subclips_ignore_index, 0, labels_subclips_action)
    labels_subclips_out = _one_hot_smooth(val_tmp)

    lam_c = lam_rows[:, None]
    labels_out = lam_c * labels_out + (1.0 - lam_c) * labels_out[partner]
    lam_s = lam_rows[:, None, None]
    labels_subclips_out = (
        lam_s * labels_subclips_out + (1.0 - lam_s) * labels_subclips_out[partner]
    )
    return (x_out, labels_out, labels_subclips_out, subclips_ignore_index)


# speedup vs baseline: 1.0250x; 1.0250x over previous
import functools

import jax
import jax.numpy as jnp
from jax.experimental import pallas as pl
from jax.experimental.pallas import tpu as pltpu

_NUM_CLASSES = 400
_LABEL_SMOOTH = 0.1
_ALPHA = 1.0
_IGNORE_CLS = -1
_B = 16


def _mix_x_body(partner_ref, lam_ref, x_ref, o_ref):
    for i in range(_B):
        lam = lam_ref[i]
        p = partner_ref[i]
        xi = x_ref[pl.ds(i, 1)]
        xp = x_ref[pl.ds(p, 1)]
        o_ref[pl.ds(i, 1)] = xi * lam + xp * (1.0 - lam)


def _mix_x(x, partner, lam_rows, tchunk):
    b, c, t, h, w = x.shape
    grid_spec = pltpu.PrefetchScalarGridSpec(
        num_scalar_prefetch=2,
        grid=(c, t // tchunk),
        in_specs=[
            pl.BlockSpec((b, 1, tchunk, h, w),
                         lambda j, k, p, l: (0, j, k, 0, 0)),
        ],
        out_specs=pl.BlockSpec((b, 1, tchunk, h, w),
                               lambda j, k, p, l: (0, j, k, 0, 0)),
    )
    return pl.pallas_call(
        _mix_x_body,
        grid_spec=grid_spec,
        out_shape=jax.ShapeDtypeStruct(x.shape, jnp.float32),
        compiler_params=pltpu.CompilerParams(
            dimension_semantics=("arbitrary", "arbitrary"),
        ),
    )(partner, lam_rows, x)


def _one_hot_smooth(t):
    nt = _LABEL_SMOOTH / _NUM_CLASSES
    tv = 1.0 - _LABEL_SMOOTH + nt
    hot = jax.nn.one_hot(jnp.squeeze(t, axis=-1), _NUM_CLASSES, dtype=jnp.float32)
    return jnp.where(hot > 0.5, jnp.float32(tv), jnp.float32(nt))


def kernel(x_video_rgb, labels_action, labels_subclips_action):
    ts = jnp.squeeze(labels_subclips_action, axis=-1)
    mask = jnp.all(ts != _IGNORE_CLS, axis=-1)
    k = jnp.sum(mask)
    no_mix = k <= 1
    order = jnp.argsort(jnp.logical_not(mask), stable=True)
    rank = jnp.cumsum(mask) - 1
    partner = order[jnp.clip(k - 1 - rank, 0, _B - 1)].astype(jnp.int32)
    lam = jax.random.beta(jax.random.key(1), _ALPHA, _ALPHA)
    mix_on = mask & jnp.logical_not(no_mix)
    lam_rows = jnp.where(mix_on, lam, 1.0).astype(jnp.float32)

    x_out = _mix_x(x_video_rgb, partner, lam_rows, tchunk=4)

    labels_out = _one_hot_smooth(labels_action)
    subclips_ignore_index = labels_subclips_action == _IGNORE_CLS
    val_tmp = jnp.where(subclips_ignore_index, 0, labels_subclips_action)
    labels_subclips_out = _one_hot_smooth(val_tmp)

    lam_c = lam_rows[:, None]
    labels_out = lam_c * labels_out + (1.0 - lam_c) * labels_out[partner]
    lam_s = lam_rows[:, None, None]
    labels_subclips_out = (
        lam_s * labels_subclips_out + (1.0 - lam_s) * labels_subclips_out[partner]
    )
    return (x_out, labels_out, labels_subclips_out, subclips_ignore_index)
